# Initial kernel scaffold; baseline (speedup 1.0000x reference)
import jax
import jax.numpy as jnp
from jax.experimental import pallas as pl


def kernel(xyz, features):
    raise NotImplementedError("write your pallas kernel here")


# speedup vs baseline: 8.0265x; 8.0265x over previous
import functools

import jax
import jax.numpy as jnp
from jax import lax
from jax.experimental import pallas as pl
from jax.experimental.pallas import tpu as pltpu
from jax.experimental.pallas import tpu_sc as plsc

_S = 512
_KNN = 32


def _fps_body(xyz_ref, far0_ref, idx_ref, nxyz_ref):
    X = xyz_ref[:, 0, :]
    Y = xyz_ref[:, 1, :]
    Z = xyz_ref[:, 2, :]
    B, N = X.shape
    iota_n = lax.broadcasted_iota(jnp.int32, (B, N), 1)
    iota_s = lax.broadcasted_iota(jnp.int32, (B, _S), 1)

    def body(i, st):
        dist, far, idx_acc, cx_acc, cy_acc, cz_acc = st
        sel = iota_n == far
        cx = jnp.sum(jnp.where(sel, X, 0.0), axis=1, keepdims=True)
        cy = jnp.sum(jnp.where(sel, Y, 0.0), axis=1, keepdims=True)
        cz = jnp.sum(jnp.where(sel, Z, 0.0), axis=1, keepdims=True)
        col = iota_s == i
        idx_acc = jnp.where(col, far, idx_acc)
        cx_acc = jnp.where(col, cx, cx_acc)
        cy_acc = jnp.where(col, cy, cy_acc)
        cz_acc = jnp.where(col, cz, cz_acc)
        dx = X - cx
        dy = Y - cy
        dz = Z - cz
        d = (dx * dx + dy * dy) + dz * dz
        dist = jnp.minimum(dist, d)
        m = jnp.max(dist, axis=1, keepdims=True)
        far = jnp.min(jnp.where(dist == m, iota_n, N), axis=1, keepdims=True)
        far = far.astype(jnp.int32)
        return (dist, far, idx_acc, cx_acc, cy_acc, cz_acc)

    dist0 = jnp.full((B, N), 1e10, dtype=jnp.float32)
    far0 = far0_ref[...]
    zero_s = jnp.zeros((B, _S), dtype=jnp.float32)
    izero_s = jnp.zeros((B, _S), dtype=jnp.int32)
    _, _, idx_acc, cx_acc, cy_acc, cz_acc = lax.fori_loop(
        0, _S, body, (dist0, far0, izero_s, zero_s, zero_s, zero_s)
    )
    idx_ref[...] = idx_acc
    nxyz_ref[:, 0, :] = cx_acc
    nxyz_ref[:, 1, :] = cy_acc
    nxyz_ref[:, 2, :] = cz_acc


def _fps(xyz, far0):
    B, _, N = xyz.shape
    return pl.pallas_call(
        _fps_body,
        out_shape=(
            jax.ShapeDtypeStruct((B, _S), jnp.int32),
            jax.ShapeDtypeStruct((B, 3, _S), jnp.float32),
        ),
    )(xyz, far0)


def _knn_body(nxyz_ref, xyz_ref, out_ref):
    a3 = nxyz_ref[0]
    xb = xyz_ref[0]
    R = a3.shape[1]
    N = xb.shape[1]
    inner = lax.dot_general(
        a3, xb, (((0,), (0,)), ((), ())), preferred_element_type=jnp.float32
    )
    sq_a = (a3[0] * a3[0] + a3[1] * a3[1]) + a3[2] * a3[2]
    sq_b = (xb[0] * xb[0] + xb[1] * xb[1]) + xb[2] * xb[2]
    d2 = (sq_a[:, None] + sq_b[None, :]) - 2.0 * inner
    d = jnp.sqrt(jnp.maximum(d2, 0.0))

    iota_n = lax.broadcasted_iota(jnp.int32, (R, N), 1)
    iota_k = lax.broadcasted_iota(jnp.int32, (R, _KNN), 1)

    def body(r, st):
        d_cur, acc = st
        m = jnp.min(d_cur, axis=1, keepdims=True)
        idx = jnp.min(jnp.where(d_cur == m, iota_n, N), axis=1, keepdims=True)
        acc = jnp.where(iota_k == r, idx, acc)
        d_cur = jnp.where(iota_n == idx, jnp.float32(jnp.inf), d_cur)
        return (d_cur, acc)

    _, acc = lax.fori_loop(0, _KNN, body, (d, jnp.zeros((R, _KNN), jnp.int32)))
    out_ref[0] = acc


def _knn(new_xyz, xyz):
    B, _, N = xyz.shape
    R = 128
    grid = (B, _S // R)
    return pl.pallas_call(
        _knn_body,
        grid=grid,
        in_specs=[
            pl.BlockSpec((1, 3, R), lambda b, s: (b, 0, s)),
            pl.BlockSpec((1, 3, N), lambda b, s: (b, 0, 0)),
        ],
        out_specs=pl.BlockSpec((1, R, _KNN), lambda b, s: (b, s, 0)),
        out_shape=jax.ShapeDtypeStruct((B, _S, _KNN), jnp.int32),
    )(new_xyz, xyz)


def _gather_sc(comb, knn_flat, cen):
    B, C, N = comb.shape
    SK = _S * _KNN
    mesh = plsc.VectorSubcoreMesh(core_axis_name="c", subcore_axis_name="s")

    @functools.partial(
        pl.kernel,
        mesh=mesh,
        compiler_params=pltpu.CompilerParams(needs_layout_passes=False),
        out_type=jax.ShapeDtypeStruct((B, C, SK), jnp.float32),
        scratch_types=[
            pltpu.VMEM((SK,), jnp.int32),
            pltpu.VMEM((N,), jnp.float32),
            pltpu.VMEM((SK,), jnp.float32),
            pltpu.VMEM((SK,), jnp.float32),
        ],
    )
    def k(comb_hbm, knn_hbm, cen_hbm, out_hbm, idx_v, tab_v, out_v, cen_v):
        cid = lax.axis_index("c")
        sid = lax.axis_index("s")
        w = sid * 2 + cid
        b = w % B
        r = w // B
        pltpu.sync_copy(knn_hbm.at[b], idx_v)

        def chan_body(j, carry):
            c = r + 4 * j

            @pl.when(c < C)
            def _():
                pltpu.sync_copy(comb_hbm.at[b, c], tab_v)

                @pl.when(c < C - 3)
                def _():
                    def g_body(t, carry2):
                        iv = idx_v[pl.ds(t * 16, 16)]
                        out_v[pl.ds(t * 16, 16)] = plsc.load_gather(tab_v, [iv])
                        return carry2

                    lax.fori_loop(0, SK // 16, g_body, 0)

                @pl.when(c >= C - 3)
                def _():
                    pltpu.sync_copy(cen_hbm.at[b, c - (C - 3)], cen_v)

                    def g_body(t, carry2):
                        sl = pl.ds(t * 16, 16)
                        iv = idx_v[sl]
                        out_v[sl] = plsc.load_gather(tab_v, [iv]) - cen_v[sl]
                        return carry2

                    lax.fori_loop(0, SK // 16, g_body, 0)

                pltpu.sync_copy(out_v, out_hbm.at[b, c])

            return carry

        lax.fori_loop(0, (C + 3) // 4, chan_body, 0)

    return k(comb, knn_flat, cen)


def kernel(xyz, features):
    B, _, N = xyz.shape
    far0 = jax.random.randint(jax.random.key(1), (B,), 0, N)
    far0 = far0.astype(jnp.int32).reshape(B, 1)

    sample_idx, new_xyz = _fps(xyz, far0)
    knn_idx = _knn(new_xyz, xyz)

    comb = jnp.concatenate([features, xyz], axis=1)
    cen = jnp.broadcast_to(
        new_xyz[:, :, :, None], (B, 3, _S, _KNN)
    ).reshape(B, 3, _S * _KNN)
    knn_flat = knn_idx.reshape(B, _S * _KNN)

    nf = _gather_sc(comb, knn_flat, cen).reshape(B, 67, _S, _KNN)
    return (new_xyz, nf)


# speedup vs baseline: 11.5641x; 1.4407x over previous
import functools

import jax
import jax.numpy as jnp
from jax import lax
from jax.experimental import pallas as pl
from jax.experimental.pallas import tpu as pltpu
from jax.experimental.pallas import tpu_sc as plsc

_S = 512
_KNN = 32


def _fps_body(xyz_ref, far0_ref, idx_ref, nxyz_ref):
    X = xyz_ref[:, 0, :]
    Y = xyz_ref[:, 1, :]
    Z = xyz_ref[:, 2, :]
    B, N = X.shape
    iota_n = lax.broadcasted_iota(jnp.int32, (B, N), 1)
    iota_s = lax.broadcasted_iota(jnp.int32, (B, _S), 1)

    def body(i, st):
        dist, far, idx_acc, cx_acc, cy_acc, cz_acc = st
        sel = iota_n == far
        cx = jnp.sum(jnp.where(sel, X, 0.0), axis=1, keepdims=True)
        cy = jnp.sum(jnp.where(sel, Y, 0.0), axis=1, keepdims=True)
        cz = jnp.sum(jnp.where(sel, Z, 0.0), axis=1, keepdims=True)
        col = iota_s == i
        idx_acc = jnp.where(col, far, idx_acc)
        cx_acc = jnp.where(col, cx, cx_acc)
        cy_acc = jnp.where(col, cy, cy_acc)
        cz_acc = jnp.where(col, cz, cz_acc)
        dx = X - cx
        dy = Y - cy
        dz = Z - cz
        d = (dx * dx + dy * dy) + dz * dz
        dist = jnp.minimum(dist, d)
        m = jnp.max(dist, axis=1, keepdims=True)
        far = jnp.min(jnp.where(dist == m, iota_n, N), axis=1, keepdims=True)
        far = far.astype(jnp.int32)
        return (dist, far, idx_acc, cx_acc, cy_acc, cz_acc)

    dist0 = jnp.full((B, N), 1e10, dtype=jnp.float32)
    far0 = far0_ref[...]
    zero_s = jnp.zeros((B, _S), dtype=jnp.float32)
    izero_s = jnp.zeros((B, _S), dtype=jnp.int32)
    _, _, idx_acc, cx_acc, cy_acc, cz_acc = lax.fori_loop(
        0, _S, body, (dist0, far0, izero_s, zero_s, zero_s, zero_s)
    )
    idx_ref[...] = idx_acc
    nxyz_ref[:, 0, :] = cx_acc
    nxyz_ref[:, 1, :] = cy_acc
    nxyz_ref[:, 2, :] = cz_acc


def _fps(xyz, far0):
    B, _, N = xyz.shape
    return pl.pallas_call(
        _fps_body,
        out_shape=(
            jax.ShapeDtypeStruct((B, _S), jnp.int32),
            jax.ShapeDtypeStruct((B, 3, _S), jnp.float32),
        ),
    )(xyz, far0)


def _dist_body(nxyz_ref, xyz_ref, d_ref, tau_ref):
    a3 = nxyz_ref[0]
    xb = xyz_ref[0]
    R = a3.shape[1]
    N = xb.shape[1]
    inner = lax.dot_general(
        a3, xb, (((0,), (0,)), ((), ())), preferred_element_type=jnp.float32
    )
    sq_a = (a3[0] * a3[0] + a3[1] * a3[1]) + a3[2] * a3[2]
    sq_b = (xb[0] * xb[0] + xb[1] * xb[1]) + xb[2] * xb[2]
    d2 = (sq_a[:, None] + sq_b[None, :]) - 2.0 * inner
    d = jnp.sqrt(jnp.maximum(d2, 0.0))
    d_ref[0] = d

    CH = 128
    NCH = N // CH
    iota_c = lax.broadcasted_iota(jnp.int32, (R, NCH), 1)
    inf = jnp.float32(jnp.inf)
    M = jnp.full((R, NCH), inf, jnp.float32)
    for c in range(NCH):
        mc = jnp.min(d[:, c * CH:(c + 1) * CH], axis=1, keepdims=True)
        M = jnp.where(iota_c == c, mc, M)

    def body(r, st):
        M_cur, _tau = st
        m = jnp.min(M_cur, axis=1, keepdims=True)
        p = jnp.min(jnp.where(M_cur == m, iota_c, NCH), axis=1, keepdims=True)
        M_cur = jnp.where(iota_c == p, inf, M_cur)
        return (M_cur, m)

    _, tau = lax.fori_loop(0, _KNN, body, (M, jnp.zeros((R, 1), jnp.float32)))
    tau_ref[0] = tau


def _dist(new_xyz, xyz):
    B, _, N = xyz.shape
    R = 128
    grid = (B, _S // R)
    return pl.pallas_call(
        _dist_body,
        grid=grid,
        in_specs=[
            pl.BlockSpec((1, 3, R), lambda b, s: (b, 0, s)),
            pl.BlockSpec((1, 3, N), lambda b, s: (b, 0, 0)),
        ],
        out_specs=[
            pl.BlockSpec((1, R, N), lambda b, s: (b, s, 0)),
            pl.BlockSpec((1, R, 1), lambda b, s: (b, s, 0)),
        ],
        out_shape=[
            jax.ShapeDtypeStruct((B, _S, N), jnp.float32),
            jax.ShapeDtypeStruct((B, _S, 1), jnp.float32),
        ],
    )(new_xyz, xyz)


def _splat_i32(x):
    return jnp.full((16,), 1, jnp.int32) * x


def _select_sc(d_all, tau):
    BS, N = d_all.shape
    NW = 32
    RW = BS // NW
    NV = N // 16
    inf = jnp.float32(jnp.inf)
    mesh = plsc.VectorSubcoreMesh(core_axis_name="c", subcore_axis_name="s")

    @functools.partial(
        pl.kernel,
        mesh=mesh,
        compiler_params=pltpu.CompilerParams(needs_layout_passes=False),
        out_type=jax.ShapeDtypeStruct((BS, _KNN), jnp.int32),
        scratch_types=[
            pltpu.VMEM((N,), jnp.float32),
            pltpu.VMEM((N + 16,), jnp.float32),
            pltpu.VMEM((N + 16,), jnp.int32),
            pltpu.VMEM((RW + 16,), jnp.float32),
            pltpu.VMEM((RW, _KNN), jnp.int32),
        ],
    )
    def k(d_hbm, tau_hbm, out_hbm, d_row, cand_v, cand_i, tau_v, out_rows):
        cid = lax.axis_index("c")
        sid = lax.axis_index("s")
        w = sid * 2 + cid
        base_row = w * RW
        pltpu.sync_copy(tau_hbm.at[pl.ds(base_row, RW)], tau_v.at[pl.ds(0, RW)])
        iota16 = lax.iota(jnp.int32, 16)

        def row_body(r, carry):
            pltpu.sync_copy(d_hbm.at[base_row + r], d_row)
            tval = tau_v[pl.ds(r, 16)][0]
            tvec = jnp.full((16,), tval, jnp.float32)

            def scan_body(i, off):
                v = d_row[pl.ds(i * 16, 16)]
                msk = v <= tvec
                plsc.store_compressed(cand_v.at[pl.ds(off, 16)], v, mask=msk)
                iv = iota16 + _splat_i32(i * 16)
                plsc.store_compressed(cand_i.at[pl.ds(off, 16)], iv, mask=msk)
                return off + jnp.sum(msk.astype(jnp.int32))

            C = lax.fori_loop(0, NV, scan_body, jnp.int32(0))
            nvec = (C + 15) // 16
            c_splat = _splat_i32(C)
            big = _splat_i32(jnp.int32(1 << 30))

            def round_body(r2, carry2):
                acc_lo, acc_hi = carry2

                def sel_body(tv, st):
                    av, ai, ap = st
                    b16 = tv * 16
                    pos = iota16 + _splat_i32(b16)
                    v = cand_v[pl.ds(b16, 16)]
                    vi = cand_i[pl.ds(b16, 16)]
                    v = jnp.where(pos < c_splat, v, inf)
                    upd = v < av
                    av = jnp.where(upd, v, av)
                    ai = jnp.where(upd, vi, ai)
                    ap = jnp.where(upd, pos, ap)
                    return (av, ai, ap)

                av0 = jnp.full((16,), inf, jnp.float32)
                az = jnp.zeros((16,), jnp.int32)
                av, ai, ap = lax.fori_loop(0, nvec, sel_body, (av0, az, az))
                m = jnp.min(av)
                tie = av == jnp.full((16,), m, jnp.float32)
                idxsel = jnp.min(jnp.where(tie, ai, big))
                psel = jnp.min(
                    jnp.where(tie & (ai == _splat_i32(idxsel)), ap, big)
                )
                isel_spl = _splat_i32(idxsel)
                acc_lo = jnp.where(iota16 == _splat_i32(r2), isel_spl, acc_lo)
                acc_hi = jnp.where(
                    iota16 == _splat_i32(r2 - 16), isel_spl, acc_hi
                )
                vbase = (psel // 16) * 16
                vec = cand_v[pl.ds(vbase, 16)]
                vec = jnp.where(
                    iota16 + _splat_i32(vbase) == _splat_i32(psel), inf, vec
                )
                cand_v[pl.ds(vbase, 16)] = vec
                return (acc_lo, acc_hi)

            az0 = jnp.zeros((16,), jnp.int32)
            acc_lo, acc_hi = lax.fori_loop(0, _KNN, round_body, (az0, az0))
            out_rows[r, pl.ds(0, 16)] = acc_lo
            out_rows[r, pl.ds(16, 16)] = acc_hi
            return carry

        lax.fori_loop(0, RW, row_body, 0)
        pltpu.sync_copy(out_rows, out_hbm.at[pl.ds(base_row, RW)])

    return k(d_all, tau)


def _gather_sc(comb, knn_flat, cen):
    B, C, N = comb.shape
    SK = _S * _KNN
    mesh = plsc.VectorSubcoreMesh(core_axis_name="c", subcore_axis_name="s")

    @functools.partial(
        pl.kernel,
        mesh=mesh,
        compiler_params=pltpu.CompilerParams(needs_layout_passes=False),
        out_type=jax.ShapeDtypeStruct((B, C, SK), jnp.float32),
        scratch_types=[
            pltpu.VMEM((SK,), jnp.int32),
            pltpu.VMEM((N,), jnp.float32),
            pltpu.VMEM((SK,), jnp.float32),
            pltpu.VMEM((SK,), jnp.float32),
        ],
    )
    def k(comb_hbm, knn_hbm, cen_hbm, out_hbm, idx_v, tab_v, out_v, cen_v):
        cid = lax.axis_index("c")
        sid = lax.axis_index("s")
        w = sid * 2 + cid
        b = w % B
        r = w // B
        pltpu.sync_copy(knn_hbm.at[b], idx_v)

        def chan_body(j, carry):
            c = r + 4 * j

            @pl.when(c < C)
            def _():
                pltpu.sync_copy(comb_hbm.at[b, c], tab_v)

                @pl.when(c < C - 3)
                def _():
                    def g_body(t, carry2):
                        iv = idx_v[pl.ds(t * 16, 16)]
                        out_v[pl.ds(t * 16, 16)] = plsc.load_gather(tab_v, [iv])
                        return carry2

                    lax.fori_loop(0, SK // 16, g_body, 0)

                @pl.when(c >= C - 3)
                def _():
                    pltpu.sync_copy(cen_hbm.at[b, c - (C - 3)], cen_v)

                    def g_body(t, carry2):
                        sl = pl.ds(t * 16, 16)
                        iv = idx_v[sl]
                        out_v[sl] = plsc.load_gather(tab_v, [iv]) - cen_v[sl]
                        return carry2

                    lax.fori_loop(0, SK // 16, g_body, 0)

                pltpu.sync_copy(out_v, out_hbm.at[b, c])

            return carry

        lax.fori_loop(0, (C + 3) // 4, chan_body, 0)

    return k(comb, knn_flat, cen)


def kernel(xyz, features):
    B, _, N = xyz.shape
    far0 = jax.random.randint(jax.random.key(1), (B,), 0, N)
    far0 = far0.astype(jnp.int32).reshape(B, 1)

    sample_idx, new_xyz = _fps(xyz, far0)
    d_all, tau = _dist(new_xyz, xyz)
    knn_rows = _select_sc(
        d_all.reshape(B * _S, N), tau.reshape(B * _S)
    )

    comb = jnp.concatenate([features, xyz], axis=1)
    cen = jnp.broadcast_to(
        new_xyz[:, :, :, None], (B, 3, _S, _KNN)
    ).reshape(B, 3, _S * _KNN)
    knn_flat = knn_rows.reshape(B, _S * _KNN)

    nf = _gather_sc(comb, knn_flat, cen).reshape(B, 67, _S, _KNN)
    return (new_xyz, nf)
